# Initial kernel scaffold; baseline (speedup 1.0000x reference)
import jax
import jax.numpy as jnp
from jax.experimental import pallas as pl


def kernel(pose0_position, pose0_rotation, pose1_position, pose1_rotation, emb_pose0_position, emb_pose0_rotation, emb_pose1_position, emb_pose1_rotation, W, b):
    raise NotImplementedError("write your pallas kernel here")


# speedup vs baseline: 5.6586x; 5.6586x over previous
import functools

import jax
import jax.numpy as jnp
from jax import lax
from jax.experimental import pallas as pl
from jax.experimental.pallas import tpu as pltpu
from jax.experimental.pallas import tpu_sc as plsc

B, T, V, D, OUT = 16384, 20, 1000, 32, 128
N = B * T
NP = 4


def _fuse_tables_body(e0, e1, e2, e3, w, bvec, f0, f1, f2, f3):
    embs = (e0, e1, e2, e3)
    outs = (f0, f1, f2, f3)
    for p in range(NP):
        fp = jnp.dot(embs[p][...], w[pl.ds(p * D, D), :],
                     preferred_element_type=jnp.float32)
        if p == 0:
            fp = fp + bvec[...]
        outs[p][...] = fp


@jax.jit
def _fuse_tables(e0, e1, e2, e3, w, bvec):
    return pl.pallas_call(
        _fuse_tables_body,
        out_shape=[jax.ShapeDtypeStruct((V, OUT), jnp.float32)] * NP,
    )(e0, e1, e2, e3, w, bvec)


def _make_gather_sum(nc, ns, nlanes):
    nw = nc * ns
    per_w = N // nw
    C = 128
    nchunk = per_w // C
    nseg = OUT // nlanes

    mesh = plsc.VectorSubcoreMesh(core_axis_name="c", subcore_axis_name="s")

    @functools.partial(
        pl.kernel,
        mesh=mesh,
        out_type=jax.ShapeDtypeStruct((N, OUT), jnp.float32),
        scratch_types=[
            pltpu.VMEM((NP, per_w), jnp.int32),
            pltpu.VMEM((NP, C, OUT), jnp.float32),
            pltpu.SemaphoreType.DMA,
        ],
    )
    def gather_sum(f0, f1, f2, f3, idx_hbm, out_hbm, idx_v, rows_v, sem):
        wid = lax.axis_index("s") * nc + lax.axis_index("c")
        base = wid * per_w
        for p in range(NP):
            pltpu.sync_copy(idx_hbm.at[p, pl.ds(base, per_w)], idx_v.at[p])
        tables = (f0, f1, f2, f3)

        def chunk(c, carry):
            off = c * C
            cps = [
                pltpu.async_copy(
                    tables[p].at[idx_v.at[p, pl.ds(off, C)]],
                    rows_v.at[p], sem)
                for p in range(NP)
            ]
            for cp in cps:
                cp.wait()

            def row(r, carry2):
                for j in range(nseg):
                    sl = pl.ds(j * nlanes, nlanes)
                    acc = (rows_v[0, r, sl] + rows_v[1, r, sl]
                           + rows_v[2, r, sl] + rows_v[3, r, sl])
                    rows_v[0, r, sl] = acc
                return carry2

            lax.fori_loop(0, C, row, 0)
            pltpu.sync_copy(rows_v.at[0], out_hbm.at[pl.ds(base + off, C)])
            return carry

        lax.fori_loop(0, nchunk, chunk, 0)

    return gather_sum


def kernel(pose0_position, pose0_rotation, pose1_position, pose1_rotation,
           emb_pose0_position, emb_pose0_rotation, emb_pose1_position,
           emb_pose1_rotation, W, b):
    info = plsc.get_sparse_core_info()
    f0, f1, f2, f3 = _fuse_tables(
        emb_pose0_position, emb_pose0_rotation, emb_pose1_position,
        emb_pose1_rotation, W, b.reshape(1, OUT))
    idx = jnp.stack([
        pose0_position.reshape(N), pose0_rotation.reshape(N),
        pose1_position.reshape(N), pose1_rotation.reshape(N)])
    gather_sum = _make_gather_sum(info.num_cores, info.num_subcores,
                                  info.num_lanes)
    out = gather_sum(f0, f1, f2, f3, idx)
    return out.reshape(B, T, OUT)


# speedup vs baseline: 7.1345x; 1.2608x over previous
import functools

import jax
import jax.numpy as jnp
from jax import lax
from jax.experimental import pallas as pl
from jax.experimental.pallas import tpu as pltpu
from jax.experimental.pallas import tpu_sc as plsc

B, T, V, D, OUT = 16384, 20, 1000, 32, 128
N = B * T
NP = 4


def _fuse_tables_body(e0, e1, e2, e3, w, bvec, f0, f1, f2, f3):
    embs = (e0, e1, e2, e3)
    outs = (f0, f1, f2, f3)
    for p in range(NP):
        fp = jnp.dot(embs[p][...], w[pl.ds(p * D, D), :],
                     preferred_element_type=jnp.float32)
        if p == 0:
            fp = fp + bvec[...]
        outs[p][...] = fp


@jax.jit
def _fuse_tables(e0, e1, e2, e3, w, bvec):
    return pl.pallas_call(
        _fuse_tables_body,
        out_shape=[jax.ShapeDtypeStruct((V, OUT), jnp.float32)] * NP,
    )(e0, e1, e2, e3, w, bvec)


def _make_gather_sum(nc, ns, nlanes):
    nw = nc * ns
    per_w = N // nw
    C = 64
    nchunk = per_w // C
    nseg = OUT // nlanes
    assert nchunk % 2 == 0 and nchunk >= 4

    mesh = plsc.VectorSubcoreMesh(core_axis_name="c", subcore_axis_name="s")

    @functools.partial(
        pl.kernel,
        mesh=mesh,
        out_type=jax.ShapeDtypeStruct((N, OUT), jnp.float32),
        scratch_types=[
            pltpu.VMEM((NP, per_w), jnp.int32),
            pltpu.VMEM((2, NP, C, OUT), jnp.float32),
            pltpu.VMEM((2, C, OUT), jnp.float32),
            pltpu.SemaphoreType.DMA,
            pltpu.SemaphoreType.DMA,
            pltpu.SemaphoreType.DMA,
            pltpu.SemaphoreType.DMA,
        ],
    )
    def gather_sum(f0, f1, f2, f3, idx_hbm, out_hbm,
                   idx_v, rows_v, acc_v, gsem0, gsem1, wsem0, wsem1):
        wid = lax.axis_index("s") * nc + lax.axis_index("c")
        base = wid * per_w
        for p in range(NP):
            pltpu.sync_copy(idx_hbm.at[p, pl.ds(base, per_w)], idx_v.at[p])
        tables = (f0, f1, f2, f3)
        gsems = (gsem0, gsem1)
        wsems = (wsem0, wsem1)

        def issue_g(c, bf):
            for p in range(NP):
                pltpu.async_copy(
                    tables[p].at[idx_v.at[p, pl.ds(c * C, C)]],
                    rows_v.at[bf, p], gsems[bf])

        def wait_g(bf):
            for p in range(NP):
                pltpu.make_async_copy(
                    tables[p].at[idx_v.at[p, pl.ds(0, C)]],
                    rows_v.at[bf, p], gsems[bf]).wait()

        def do_adds(bf):
            def row(r, carry):
                for j in range(nseg):
                    sl = pl.ds(j * nlanes, nlanes)
                    acc_v[bf, r, sl] = (
                        rows_v[bf, 0, r, sl] + rows_v[bf, 1, r, sl]
                        + rows_v[bf, 2, r, sl] + rows_v[bf, 3, r, sl])
                return carry
            lax.fori_loop(0, C, row, 0)

        def issue_w(c, bf):
            pltpu.async_copy(acc_v.at[bf],
                             out_hbm.at[pl.ds(base + c * C, C)], wsems[bf])

        def wait_w(bf):
            pltpu.make_async_copy(acc_v.at[bf],
                                  out_hbm.at[pl.ds(base, C)],
                                  wsems[bf]).wait()

        issue_g(0, 0)
        issue_g(1, 1)
        for bf in (0, 1):
            wait_g(bf)
            do_adds(bf)
            issue_w(bf, bf)
            issue_g(bf + 2, bf)

        def pair(cc, carry):
            c0 = 2 * cc
            for bf in (0, 1):
                c = c0 + bf
                wait_g(bf)
                wait_w(bf)
                do_adds(bf)
                issue_w(c, bf)
                issue_g(c + 2, bf)
            return carry

        lax.fori_loop(1, nchunk // 2 - 1, pair, 0)

        for bf in (0, 1):
            wait_g(bf)
            wait_w(bf)
            do_adds(bf)
            issue_w(nchunk - 2 + bf, bf)
        for bf in (0, 1):
            wait_w(bf)

    return gather_sum


def kernel(pose0_position, pose0_rotation, pose1_position, pose1_rotation,
           emb_pose0_position, emb_pose0_rotation, emb_pose1_position,
           emb_pose1_rotation, W, b):
    info = plsc.get_sparse_core_info()
    f0, f1, f2, f3 = _fuse_tables(
        emb_pose0_position, emb_pose0_rotation, emb_pose1_position,
        emb_pose1_rotation, W, b.reshape(1, OUT))
    idx = jnp.stack([
        pose0_position.reshape(N), pose0_rotation.reshape(N),
        pose1_position.reshape(N), pose1_rotation.reshape(N)])
    gather_sum = _make_gather_sum(info.num_cores, info.num_subcores,
                                  info.num_lanes)
    out = gather_sum(f0, f1, f2, f3, idx)
    return out.reshape(B, T, OUT)


# speedup vs baseline: 7.2792x; 1.0203x over previous
import functools

import jax
import jax.numpy as jnp
from jax import lax
from jax.experimental import pallas as pl
from jax.experimental.pallas import tpu as pltpu
from jax.experimental.pallas import tpu_sc as plsc

B, T, V, D, OUT = 16384, 20, 1000, 32, 128
N = B * T
NP = 4


def _fuse_tables_body(e0, e1, e2, e3, w, bvec, f0, f1, f2, f3):
    embs = (e0, e1, e2, e3)
    outs = (f0, f1, f2, f3)
    for p in range(NP):
        fp = jnp.dot(embs[p][...], w[pl.ds(p * D, D), :],
                     preferred_element_type=jnp.float32)
        if p == 0:
            fp = fp + bvec[...]
        outs[p][...] = fp


@jax.jit
def _fuse_tables(e0, e1, e2, e3, w, bvec):
    return pl.pallas_call(
        _fuse_tables_body,
        out_shape=[jax.ShapeDtypeStruct((V, OUT), jnp.float32)] * NP,
    )(e0, e1, e2, e3, w, bvec)


def _make_gather_sum(nc, ns, nlanes):
    nw = nc * ns
    per_w = N // nw
    C = 64
    nchunk = per_w // C
    nseg = OUT // nlanes
    assert nchunk % 2 == 0 and nchunk >= 4

    mesh = plsc.VectorSubcoreMesh(core_axis_name="c", subcore_axis_name="s")

    @functools.partial(
        pl.kernel,
        mesh=mesh,
        out_type=jax.ShapeDtypeStruct((N, OUT), jnp.float32),
        scratch_types=[
            pltpu.VMEM((NP, per_w), jnp.int32),
            pltpu.VMEM((2, NP, C, OUT), jnp.float32),
            pltpu.VMEM((2, C, OUT), jnp.float32),
            pltpu.SemaphoreType.DMA,
            pltpu.SemaphoreType.DMA,
            pltpu.SemaphoreType.DMA,
            pltpu.SemaphoreType.DMA,
        ],
    )
    def gather_sum(f0, f1, f2, f3, i0, i1, i2, i3, out_hbm,
                   idx_v, rows_v, acc_v, gsem0, gsem1, wsem0, wsem1):
        wid = lax.axis_index("s") * nc + lax.axis_index("c")
        base = wid * per_w
        for p, ih in enumerate((i0, i1, i2, i3)):
            pltpu.sync_copy(ih.at[pl.ds(base, per_w)], idx_v.at[p])
        tables = (f0, f1, f2, f3)
        gsems = (gsem0, gsem1)
        wsems = (wsem0, wsem1)

        def issue_g(c, bf):
            for p in range(NP):
                pltpu.async_copy(
                    tables[p].at[idx_v.at[p, pl.ds(c * C, C)]],
                    rows_v.at[bf, p], gsems[bf])

        def wait_g(bf):
            for p in range(NP):
                pltpu.make_async_copy(
                    tables[p].at[idx_v.at[p, pl.ds(0, C)]],
                    rows_v.at[bf, p], gsems[bf]).wait()

        def do_adds(bf):
            def row(r, carry):
                for j in range(nseg):
                    sl = pl.ds(j * nlanes, nlanes)
                    acc_v[bf, r, sl] = (
                        rows_v[bf, 0, r, sl] + rows_v[bf, 1, r, sl]
                        + rows_v[bf, 2, r, sl] + rows_v[bf, 3, r, sl])
                return carry
            lax.fori_loop(0, C, row, 0)

        def issue_w(c, bf):
            pltpu.async_copy(acc_v.at[bf],
                             out_hbm.at[pl.ds(base + c * C, C)], wsems[bf])

        def wait_w(bf):
            pltpu.make_async_copy(acc_v.at[bf],
                                  out_hbm.at[pl.ds(base, C)],
                                  wsems[bf]).wait()

        issue_g(0, 0)
        issue_g(1, 1)
        for bf in (0, 1):
            wait_g(bf)
            do_adds(bf)
            issue_w(bf, bf)
            issue_g(bf + 2, bf)

        def pair(cc, carry):
            c0 = 2 * cc
            for bf in (0, 1):
                c = c0 + bf
                wait_g(bf)
                wait_w(bf)
                do_adds(bf)
                issue_w(c, bf)
                issue_g(c + 2, bf)
            return carry

        lax.fori_loop(1, nchunk // 2 - 1, pair, 0)

        for bf in (0, 1):
            wait_g(bf)
            wait_w(bf)
            do_adds(bf)
            issue_w(nchunk - 2 + bf, bf)
        for bf in (0, 1):
            wait_w(bf)

    return gather_sum


def kernel(pose0_position, pose0_rotation, pose1_position, pose1_rotation,
           emb_pose0_position, emb_pose0_rotation, emb_pose1_position,
           emb_pose1_rotation, W, b):
    info = plsc.get_sparse_core_info()
    f0, f1, f2, f3 = _fuse_tables(
        emb_pose0_position, emb_pose0_rotation, emb_pose1_position,
        emb_pose1_rotation, W, b.reshape(1, OUT))
    gather_sum = _make_gather_sum(info.num_cores, info.num_subcores,
                                  info.num_lanes)
    out = gather_sum(f0, f1, f2, f3,
                     pose0_position.reshape(N), pose0_rotation.reshape(N),
                     pose1_position.reshape(N), pose1_rotation.reshape(N))
    return out.reshape(B, T, OUT)


# speedup vs baseline: 9.5088x; 1.3063x over previous
import functools

import jax
import jax.numpy as jnp
from jax import lax
from jax.experimental import pallas as pl
from jax.experimental.pallas import tpu as pltpu
from jax.experimental.pallas import tpu_sc as plsc

B, T, V, D, OUT = 16384, 20, 1000, 32, 128
N = B * T
NP = 4


def _fuse_tables_body(e0, e1, e2, e3, w, bvec, f0, f1, f2, f3):
    embs = (e0, e1, e2, e3)
    outs = (f0, f1, f2, f3)
    for p in range(NP):
        fp = jnp.dot(embs[p][...], w[pl.ds(p * D, D), :],
                     preferred_element_type=jnp.float32)
        if p == 0:
            fp = fp + bvec[...]
        outs[p][...] = fp


@jax.jit
def _fuse_tables(e0, e1, e2, e3, w, bvec):
    return pl.pallas_call(
        _fuse_tables_body,
        out_shape=[jax.ShapeDtypeStruct((V, OUT), jnp.float32)] * NP,
    )(e0, e1, e2, e3, w, bvec)


def _make_gather_sum(nc, ns, nlanes):
    nw = nc * ns
    per_w = N // nw
    CB = 4
    C = CB * T
    per_wb = per_w // T
    nchunk = per_wb // CB
    nseg = OUT // nlanes
    assert per_w % T == 0 and per_wb % CB == 0 and C % 16 == 0 and C <= 128
    assert nchunk % 2 == 0 and nchunk >= 4

    mesh = plsc.VectorSubcoreMesh(core_axis_name="c", subcore_axis_name="s")

    @functools.partial(
        pl.kernel,
        mesh=mesh,
        out_type=jax.ShapeDtypeStruct((B, T, OUT), jnp.float32),
        scratch_types=[
            pltpu.VMEM((2, NP, C), jnp.int32),
            pltpu.VMEM((2, NP, C, OUT), jnp.float32),
            pltpu.VMEM((2, CB, T, OUT), jnp.float32),
            pltpu.SemaphoreType.DMA,
            pltpu.SemaphoreType.DMA,
            pltpu.SemaphoreType.DMA,
            pltpu.SemaphoreType.DMA,
            pltpu.SemaphoreType.DMA,
            pltpu.SemaphoreType.DMA,
        ],
    )
    def gather_sum(f0, f1, f2, f3, i0, i1, i2, i3, out_hbm,
                   idx_v, rows_v, acc_v,
                   gsem0, gsem1, wsem0, wsem1, isem0, isem1):
        wid = lax.axis_index("s") * nc + lax.axis_index("c")
        base = wid * per_w
        bbase = wid * per_wb
        idxs = (i0, i1, i2, i3)
        tables = (f0, f1, f2, f3)
        gsems = (gsem0, gsem1)
        wsems = (wsem0, wsem1)
        isems = (isem0, isem1)

        def issue_i(c, bf):
            for p in range(NP):
                pltpu.async_copy(idxs[p].at[pl.ds(base + c * C, C)],
                                 idx_v.at[bf, p], isems[bf])

        def wait_i(bf):
            for p in range(NP):
                pltpu.make_async_copy(idxs[p].at[pl.ds(base, C)],
                                      idx_v.at[bf, p], isems[bf]).wait()

        def issue_g(bf):
            for p in range(NP):
                pltpu.async_copy(tables[p].at[idx_v.at[bf, p]],
                                 rows_v.at[bf, p], gsems[bf])

        def wait_g(bf):
            for p in range(NP):
                pltpu.make_async_copy(tables[p].at[idx_v.at[bf, p]],
                                      rows_v.at[bf, p], gsems[bf]).wait()

        def do_adds(bf):
            for bb in range(CB):
                def row(t, carry):
                    r = bb * T + t
                    for j in range(nseg):
                        sl = pl.ds(j * nlanes, nlanes)
                        acc_v[bf, bb, t, sl] = (
                            rows_v[bf, 0, r, sl] + rows_v[bf, 1, r, sl]
                            + rows_v[bf, 2, r, sl] + rows_v[bf, 3, r, sl])
                    return carry
                lax.fori_loop(0, T, row, 0)

        def issue_w(c, bf):
            pltpu.async_copy(acc_v.at[bf],
                             out_hbm.at[pl.ds(bbase + c * CB, CB)], wsems[bf])

        def wait_w(bf):
            pltpu.make_async_copy(acc_v.at[bf],
                                  out_hbm.at[pl.ds(bbase, CB)],
                                  wsems[bf]).wait()

        issue_i(0, 0)
        issue_i(1, 1)
        for bf in (0, 1):
            wait_i(bf)
            issue_g(bf)
        for bf in (0, 1):
            wait_g(bf)
            issue_i(bf + 2, bf)
            do_adds(bf)
            issue_w(bf, bf)
            wait_i(bf)
            issue_g(bf)

        def pair(cc, carry):
            c0 = 2 * cc
            for bf in (0, 1):
                c = c0 + bf
                wait_g(bf)
                issue_i(c + 2, bf)
                wait_w(bf)
                do_adds(bf)
                issue_w(c, bf)
                wait_i(bf)
                issue_g(bf)
            return carry

        lax.fori_loop(1, nchunk // 2 - 1, pair, 0)

        for bf in (0, 1):
            wait_g(bf)
            wait_w(bf)
            do_adds(bf)
            issue_w(nchunk - 2 + bf, bf)
        for bf in (0, 1):
            wait_w(bf)

    return gather_sum


def kernel(pose0_position, pose0_rotation, pose1_position, pose1_rotation,
           emb_pose0_position, emb_pose0_rotation, emb_pose1_position,
           emb_pose1_rotation, W, b):
    info = plsc.get_sparse_core_info()
    f0, f1, f2, f3 = _fuse_tables(
        emb_pose0_position, emb_pose0_rotation, emb_pose1_position,
        emb_pose1_rotation, W, b.reshape(1, OUT))
    gather_sum = _make_gather_sum(info.num_cores, info.num_subcores,
                                  info.num_lanes)
    return gather_sum(f0, f1, f2, f3,
                      pose0_position.reshape(N), pose0_rotation.reshape(N),
                      pose1_position.reshape(N), pose1_rotation.reshape(N))


# speedup vs baseline: 9.5142x; 1.0006x over previous
import functools

import jax
import jax.numpy as jnp
from jax import lax
from jax.experimental import pallas as pl
from jax.experimental.pallas import tpu as pltpu
from jax.experimental.pallas import tpu_sc as plsc

B, T, V, D, OUT = 16384, 20, 1000, 32, 128
N = B * T
NP = 4


def _fuse_tables_body(e0, e1, e2, e3, w, bvec, f0, f1, f2, f3):
    embs = (e0, e1, e2, e3)
    outs = (f0, f1, f2, f3)
    for p in range(NP):
        fp = jnp.dot(embs[p][...], w[pl.ds(p * D, D), :],
                     preferred_element_type=jnp.float32)
        if p == 0:
            fp = fp + bvec[...]
        outs[p][...] = fp


@jax.jit
def _fuse_tables(e0, e1, e2, e3, w, bvec):
    return pl.pallas_call(
        _fuse_tables_body,
        out_shape=[jax.ShapeDtypeStruct((V, OUT), jnp.float32)] * NP,
    )(e0, e1, e2, e3, w, bvec)


def _make_gather_sum(nc, ns, nlanes):
    nw = nc * ns
    per_w = N // nw
    CB = 4
    C = CB * T
    per_wb = per_w // T
    nchunk = per_wb // CB
    nseg = OUT // nlanes
    assert per_w % T == 0 and per_wb % CB == 0 and C % 16 == 0 and C <= 128
    assert nchunk % 2 == 0 and nchunk >= 4

    mesh = plsc.VectorSubcoreMesh(core_axis_name="c", subcore_axis_name="s")

    @functools.partial(
        pl.kernel,
        mesh=mesh,
        compiler_params=pltpu.CompilerParams(use_tc_tiling_on_sc=True),
        out_type=jax.ShapeDtypeStruct((B, T, OUT), jnp.float32),
        scratch_types=[
            pltpu.VMEM((2, NP, C), jnp.int32),
            pltpu.VMEM((2, NP, C, OUT), jnp.float32),
            pltpu.VMEM((2, CB, T, OUT), jnp.float32),
            pltpu.SemaphoreType.DMA,
            pltpu.SemaphoreType.DMA,
            pltpu.SemaphoreType.DMA,
            pltpu.SemaphoreType.DMA,
            pltpu.SemaphoreType.DMA,
            pltpu.SemaphoreType.DMA,
        ],
    )
    def gather_sum(f0, f1, f2, f3, i0, i1, i2, i3, out_hbm,
                   idx_v, rows_v, acc_v,
                   gsem0, gsem1, wsem0, wsem1, isem0, isem1):
        wid = lax.axis_index("s") * nc + lax.axis_index("c")
        base = wid * per_w
        bbase = wid * per_wb
        idxs = (i0, i1, i2, i3)
        tables = (f0, f1, f2, f3)
        gsems = (gsem0, gsem1)
        wsems = (wsem0, wsem1)
        isems = (isem0, isem1)

        def issue_i(c, bf):
            for p in range(NP):
                pltpu.async_copy(idxs[p].at[pl.ds(base + c * C, C)],
                                 idx_v.at[bf, p], isems[bf])

        def wait_i(bf):
            for p in range(NP):
                pltpu.make_async_copy(idxs[p].at[pl.ds(base, C)],
                                      idx_v.at[bf, p], isems[bf]).wait()

        def issue_g(bf):
            for p in range(NP):
                pltpu.async_copy(tables[p].at[idx_v.at[bf, p]],
                                 rows_v.at[bf, p], gsems[bf])

        def wait_g(bf):
            for p in range(NP):
                pltpu.make_async_copy(tables[p].at[idx_v.at[bf, p]],
                                      rows_v.at[bf, p], gsems[bf]).wait()

        def do_adds(bf):
            for bb in range(CB):
                def row(t, carry):
                    r = bb * T + t
                    for j in range(nseg):
                        sl = pl.ds(j * nlanes, nlanes)
                        acc_v[bf, bb, t, sl] = (
                            rows_v[bf, 0, r, sl] + rows_v[bf, 1, r, sl]
                            + rows_v[bf, 2, r, sl] + rows_v[bf, 3, r, sl])
                    return carry
                lax.fori_loop(0, T, row, 0)

        def issue_w(c, bf):
            pltpu.async_copy(acc_v.at[bf],
                             out_hbm.at[pl.ds(bbase + c * CB, CB)], wsems[bf])

        def wait_w(bf):
            pltpu.make_async_copy(acc_v.at[bf],
                                  out_hbm.at[pl.ds(bbase, CB)],
                                  wsems[bf]).wait()

        issue_i(0, 0)
        issue_i(1, 1)
        for bf in (0, 1):
            wait_i(bf)
            issue_g(bf)
        for bf in (0, 1):
            wait_g(bf)
            issue_i(bf + 2, bf)
            do_adds(bf)
            issue_w(bf, bf)
            wait_i(bf)
            issue_g(bf)

        def pair(cc, carry):
            c0 = 2 * cc
            for bf in (0, 1):
                c = c0 + bf
                wait_g(bf)
                issue_i(c + 2, bf)
                wait_w(bf)
                do_adds(bf)
                issue_w(c, bf)
                wait_i(bf)
                issue_g(bf)
            return carry

        lax.fori_loop(1, nchunk // 2 - 1, pair, 0)

        for bf in (0, 1):
            wait_g(bf)
            wait_w(bf)
            do_adds(bf)
            issue_w(nchunk - 2 + bf, bf)
        for bf in (0, 1):
            wait_w(bf)

    return gather_sum


def kernel(pose0_position, pose0_rotation, pose1_position, pose1_rotation,
           emb_pose0_position, emb_pose0_rotation, emb_pose1_position,
           emb_pose1_rotation, W, b):
    info = plsc.get_sparse_core_info()
    f0, f1, f2, f3 = _fuse_tables(
        emb_pose0_position, emb_pose0_rotation, emb_pose1_position,
        emb_pose1_rotation, W, b.reshape(1, OUT))
    gather_sum = _make_gather_sum(info.num_cores, info.num_subcores,
                                  info.num_lanes)
    return gather_sum(f0, f1, f2, f3,
                      pose0_position.reshape(N), pose0_rotation.reshape(N),
                      pose1_position.reshape(N), pose1_rotation.reshape(N))
